# baseline (speedup 1.0000x reference)
import functools

import numpy as np
import jax
import jax.numpy as jnp
from jax import lax
from jax.experimental import pallas as pl
from jax.experimental.pallas import tpu as pltpu
from jax.experimental.pallas import tpu_sc as plsc

B = 1024
T = 200
BT = B * T
LD = 64
HB = B // 2
SVOCAB = (100000, 100000, 1000, 100)

NC = 2
NS = 16
NW = NC * NS
TPW = (T + NW - 1) // NW

RB = 7 * LD // 8
CB = B // 128


def _sc_all(ke0, ke1, ke2, i0, i1, i2, sefs, sidx64):
    mesh = plsc.VectorSubcoreMesh(core_axis_name="c", subcore_axis_name="s",
                                  num_cores=NC, num_subcores=NS)

    @functools.partial(
        pl.kernel,
        out_type=(
            jax.ShapeDtypeStruct((T, RB, CB, 8, 128), jnp.float32),
            jax.ShapeDtypeStruct((4 * LD * B,), jnp.float32),
        ),
        mesh=mesh,
        scratch_types=[
            pltpu.VMEM((B,), jnp.int32),
            pltpu.VMEM((HB, LD), jnp.float32),
            pltpu.VMEM((HB, LD), jnp.float32),
            pltpu.VMEM((8, 4, 8, 128), jnp.float32),
            pltpu.VMEM((B,), jnp.float32),
            pltpu.SemaphoreType.DMA,
            pltpu.SemaphoreType.DMA,
        ],
        compiler_params=pltpu.CompilerParams(use_tc_tiling_on_sc=False,
                                             needs_layout_passes=False),
    )
    def k(ke0_h, ke1_h, ke2_h, i0_h, i1_h, i2_h,
          sef0_h, sef1_h, sef2_h, sef3_h, sidx_h,
          okp_h, gs_h,
          idx_v, rows0_v, rows1_v, trans_v, srow_v, sem_g, sem_w):
        wid = lax.axis_index("s") * NC + lax.axis_index("c")
        lanes = lax.iota(jnp.int32, 16)
        rbufs = (rows0_v, rows1_v)

        for i, (tab_h, i_h) in enumerate(((ke0_h, i0_h),
                                          (ke1_h, i1_h),
                                          (ke2_h, i2_h))):
            def tbody(k_, _, tab_h=tab_h, i_h=i_h, i=i):
                t = wid + NW * k_

                @pl.when(t < T)
                def _(t=t, tab_h=tab_h, i_h=i_h, i=i):
                    pltpu.sync_copy(i_h.at[pl.ds(t * B, B)], idx_v)
                    d0 = pltpu.async_copy(
                        tab_h.at[idx_v.at[pl.ds(0, HB)]], rows0_v, sem_g)
                    for h in range(2):
                        if h == 0:
                            d0.wait()
                            d1 = pltpu.async_copy(
                                tab_h.at[idx_v.at[pl.ds(HB, HB)]],
                                rows1_v, sem_g)
                        else:
                            d1.wait()
                        rows_v = rbufs[h]

                        def lbody(l, rows_v=rows_v):
                            lb = l // 8
                            li = l - lb * 8
                            col = jnp.full((16,), l, jnp.int32)
                            for c in range(HB // 16):
                                v = plsc.load_gather(
                                    rows_v, [lanes + c * 16, col])
                                trans_v[lb, c // 8, li,
                                        pl.ds((c % 8) * 16, 16)] = v

                        plsc.parallel_loop(0, LD, 1, unroll=4)(lbody)
                        pltpu.sync_copy(
                            trans_v,
                            okp_h.at[t, pl.ds(32 + i * 8, 8),
                                     pl.ds(h * 4, 4)])

                return 0

            lax.fori_loop(0, TPW, tbody, 0)

        sj = wid // 8
        l0 = (wid % 8) * 8
        for j, sef_h in enumerate((sef0_h, sef1_h, sef2_h, sef3_h)):
            @pl.when(sj == j)
            def _(j=j, sef_h=sef_h):
                def srow_body(k_, _, j=j, sef_h=sef_h):
                    off = (j * LD + l0 + k_) * B
                    pltpu.sync_copy(sidx_h.at[pl.ds(off, B)], idx_v)
                    sdescs = []
                    for m in range(B // 128):
                        sdescs.append(pltpu.async_copy(
                            sef_h.at[idx_v.at[pl.ds(m * 128, 128)]],
                            srow_v.at[pl.ds(m * 128, 128)], sem_g))
                    for d in sdescs:
                        d.wait()
                    pltpu.sync_copy(srow_v, gs_h.at[pl.ds(off, B)])
                    return 0

                lax.fori_loop(0, 8, srow_body, 0)

    return k(ke0, ke1, ke2, i0, i1, i2, *sefs, sidx64)


def _tc_obs(xpad, wo5, bo5):
    def body(x_ref, wo_ref, bo_ref, oo_ref):
        x = x_ref[0]
        xo = jnp.broadcast_to(x[0:6][:, None, :], (6, LD, B)).reshape(6 * LD, B)
        oo_ref[0] = xo * wo_ref[...] + bo_ref[...]

    full = lambda shape: pl.BlockSpec(shape, lambda t: (0,) * len(shape))
    return pl.pallas_call(
        body,
        grid=(T,),
        in_specs=[
            pl.BlockSpec((1, 8, B), lambda t: (t, 0, 0)),
            full((6 * LD, 1)),
            full((6 * LD, 1)),
        ],
        out_specs=[pl.BlockSpec((1, 6 * LD, B), lambda t: (t, 0, 0))],
        out_shape=[jax.ShapeDtypeStruct((T, 6 * LD, B), jnp.float32)],
    )(xpad, wo5, bo5)[0]


def _tc_known_dense(okp5, xpad, wk3, bk3):
    def body(okin_ref, x_ref, wk_ref, bk_ref, ok_ref):
        del okin_ref
        x = x_ref[0]
        x4 = x[0:4].reshape(4, CB, 128)
        w5 = wk_ref[...].reshape(4, 8, 1, 8, 1)
        b5 = bk_ref[...].reshape(4, 8, 1, 8, 1)
        x5 = x4[:, None, :, None, :]
        ok_ref[0] = (x5 * w5 + b5).reshape(32, CB, 8, 128)

    full = lambda shape: pl.BlockSpec(shape, lambda t: (0,) * len(shape))
    hbm = pl.BlockSpec(memory_space=pltpu.MemorySpace.HBM)
    return pl.pallas_call(
        body,
        grid=(T,),
        in_specs=[
            hbm,
            pl.BlockSpec((1, 8, B), lambda t: (t, 0, 0)),
            full((4, 8, 8)),
            full((4, 8, 8)),
        ],
        out_specs=[
            pl.BlockSpec((1, 32, CB, 8, 128), lambda t: (t, 0, 0, 0, 0)),
        ],
        out_shape=[
            jax.ShapeDtypeStruct((T, RB, CB, 8, 128), jnp.float32),
        ],
        input_output_aliases={0: 0},
    )(okp5, xpad, wk3, bk3)[0]


def kernel(inputs, se0, se1, se2, se3, ke0, ke1, ke2, Wr, br, Wo, bo):
    inT = jnp.transpose(inputs, (1, 2, 0))
    xpad = jnp.concatenate(
        [inT[:, 0:6, :], jnp.zeros((T, 2, B), jnp.float32)], axis=1)

    kidxT = inT[:, 10:13, :].astype(jnp.int32)
    i0 = kidxT[:, 0, :].reshape(BT)
    i1 = kidxT[:, 1, :].reshape(BT)
    i2 = kidxT[:, 2, :].reshape(BT)

    sidxb = inputs[:, 0, 6:10].astype(jnp.int32)
    planes = [
        (jnp.arange(LD, dtype=jnp.int32)[:, None] * SVOCAB[j] + sidxb[None, :, j])
        for j in range(4)
    ]
    sidx64 = jnp.stack(planes, axis=0).reshape(4 * LD * B)
    sefs = [t.T.reshape(-1) for t in (se0, se1, se2, se3)]

    okp5, gs = _sc_all(ke0, ke1, ke2, i0, i1, i2, sefs, sidx64)

    oo_p = _tc_obs(xpad, Wo.reshape(6 * LD, 1), bo.reshape(6 * LD, 1))
    okp5 = _tc_known_dense(okp5, xpad, Wr.reshape(4, 8, 8), br.reshape(4, 8, 8))

    a6 = okp5.reshape(T, 7, 8, CB, 8, 128)
    known = jnp.transpose(a6, (3, 5, 0, 2, 4, 1)).reshape(B, T, LD, 7)
    static = jnp.transpose(gs.reshape(4, LD, B), (2, 0, 1))
    observed = jnp.transpose(oo_p.reshape(T, 6, LD, B), (3, 0, 2, 1))
    return (static, known, observed)


# speedup vs baseline: 1.3503x; 1.3503x over previous
import functools

import numpy as np
import jax
import jax.numpy as jnp
from jax import lax
from jax.experimental import pallas as pl
from jax.experimental.pallas import tpu as pltpu
from jax.experimental.pallas import tpu_sc as plsc

B = 1024
T = 200
BT = B * T
LD = 64
SVOCAB = (100000, 100000, 1000, 100)

NC = 2
NS = 16
NW = NC * NS

NSTAGE = 5
TS = T // NSTAGE
HB = B // 2
ROWS_S = TS * HB
CHUNK = ROWS_S // NW


def _sc_stage(with_static, ke0, ke1, ke2, idxs, sefs, sidx64):
    mesh = plsc.VectorSubcoreMesh(core_axis_name="c", subcore_axis_name="s",
                                  num_cores=NC, num_subcores=NS)
    gshape = jax.ShapeDtypeStruct((ROWS_S, 128), jnp.float32)
    out_type = [gshape, gshape, gshape]
    if with_static:
        out_type.append(jax.ShapeDtypeStruct((4 * LD * B,), jnp.float32))

    @functools.partial(
        pl.kernel,
        out_type=tuple(out_type),
        mesh=mesh,
        scratch_types=[
            pltpu.VMEM((CHUNK,), jnp.int32),
            pltpu.VMEM((CHUNK,), jnp.int32),
            pltpu.VMEM((CHUNK,), jnp.int32),
            pltpu.VMEM((CHUNK, LD), jnp.float32),
            pltpu.VMEM((CHUNK, LD), jnp.float32),
            pltpu.VMEM((CHUNK, LD), jnp.float32),
            pltpu.VMEM((B,), jnp.int32),
            pltpu.VMEM((B,), jnp.float32),
            pltpu.SemaphoreType.DMA,
            pltpu.SemaphoreType.DMA,
            pltpu.SemaphoreType.DMA,
        ],
        compiler_params=pltpu.CompilerParams(use_tc_tiling_on_sc=False),
    )
    def k(*args):
        (ke0_h, ke1_h, ke2_h, iA0_h, iB0_h, iA1_h, iB1_h, iA2_h, iB2_h,
         sef0_h, sef1_h, sef2_h, sef3_h, sidx_h) = args[:14]
        if with_static:
            g0_h, g1_h, g2_h, gs_h = args[14:18]
            scratch = args[18:]
        else:
            g0_h, g1_h, g2_h = args[14:17]
            gs_h = None
            scratch = args[17:]
        (i0_v, i1_v, i2_v, r0_v, r1_v, r2_v, sidx_v, srow_v,
         sem_i, sem_g, sem_w) = scratch
        wid = lax.axis_index("s") * NC + lax.axis_index("c")
        base = wid * CHUNK
        tabs = (ke0_h, ke1_h, ke2_h)
        gouts = (g0_h, g1_h, g2_h)
        ivs = (i0_v, i1_v, i2_v)
        rvs = (r0_v, r1_v, r2_v)

        wdescs = []
        for half, (cofs, ihs) in enumerate((
                (0, (iA0_h, iA1_h, iA2_h)),
                (LD, (iB0_h, iB1_h, iB2_h)))):
            idescs = [pltpu.async_copy(ihs[i].at[pl.ds(base, CHUNK)],
                                       ivs[i], sem_i) for i in range(3)]
            for d in idescs:
                d.wait()
            for d in wdescs:
                d.wait()
            gdescs = []
            for i in range(3):
                gdescs.append(pltpu.async_copy(
                    tabs[i].at[ivs[i]], rvs[i], sem_g))
            for d in gdescs:
                d.wait()
            wdescs = [pltpu.async_copy(
                rvs[i], gouts[i].at[pl.ds(base, CHUNK), pl.ds(cofs, LD)],
                sem_w) for i in range(3)]
        for d in wdescs:
            d.wait()

        if with_static:
            sj = wid // 8
            l0 = (wid % 8) * 8
            for j, sef_h in enumerate((sef0_h, sef1_h, sef2_h, sef3_h)):
                @pl.when(sj == j)
                def _(j=j, sef_h=sef_h):
                    def srow_body(k_, _, j=j, sef_h=sef_h):
                        off = (j * LD + l0 + k_) * B
                        pltpu.sync_copy(sidx_h.at[pl.ds(off, B)], sidx_v)
                        sdescs = []
                        for m in range(B // 128):
                            sdescs.append(pltpu.async_copy(
                                sef_h.at[sidx_v.at[pl.ds(m * 128, 128)]],
                                srow_v.at[pl.ds(m * 128, 128)], sem_g))
                        for d in sdescs:
                            d.wait()
                        pltpu.sync_copy(srow_v, gs_h.at[pl.ds(off, B)])
                        return 0

                    lax.fori_loop(0, 8, srow_body, 0)

    return k(ke0, ke1, ke2, *idxs, *sefs, sidx64)


def _tc_stage(s, prev, xpad, g0, g1, g2, wk, bk, wo, bob, eye):
    def body(*refs):
        if prev is None:
            (x_ref, g0_ref, g1_ref, g2_ref,
             wk_ref, bk_ref, wo_ref, bo_ref, eye_ref, ok_ref, oo_ref) = refs
        else:
            (_, _, x_ref, g0_ref, g1_ref, g2_ref,
             wk_ref, bk_ref, wo_ref, bo_ref, eye_ref, ok_ref, oo_ref) = refs
        x = x_ref[0]
        xk = jnp.broadcast_to(x[0:4][:, None, :], (4, LD, B)).reshape(4 * LD, B)
        ok_ref[0, 0:4 * LD, :] = xk * wk_ref[...] + bk_ref[...]
        for i, g_ref in enumerate((g0_ref, g1_ref, g2_ref)):
            g = g_ref[0]
            r0, r1 = (4 + i) * LD, (5 + i) * LD
            ok_ref[0, r0:r1, 0:HB] = lax.dot_general(
                eye_ref[...], g[:, 0:LD], (((1,), (1,)), ((), ())),
                preferred_element_type=jnp.float32)
            ok_ref[0, r0:r1, HB:B] = lax.dot_general(
                eye_ref[...], g[:, LD:128], (((1,), (1,)), ((), ())),
                preferred_element_type=jnp.float32)
        xo = jnp.broadcast_to(x[0:6][:, None, :], (6, LD, B)).reshape(6 * LD, B)
        oo_ref[0] = xo * wo_ref[...] + bo_ref[...]

    full = lambda shape: pl.BlockSpec(shape, lambda t: (0,) * len(shape))
    hbm = pl.BlockSpec(memory_space=pltpu.MemorySpace.HBM)
    alias_specs = [] if prev is None else [hbm, hbm]
    alias_args = () if prev is None else (prev[0], prev[1])
    aliases = {} if prev is None else {0: 0, 1: 1}
    return pl.pallas_call(
        body,
        grid=(TS,),
        in_specs=alias_specs + [
            pl.BlockSpec((1, 8, B), lambda t, s=s: (s * TS + t, 0, 0)),
            pl.BlockSpec((1, HB, 128), lambda t: (t, 0, 0)),
            pl.BlockSpec((1, HB, 128), lambda t: (t, 0, 0)),
            pl.BlockSpec((1, HB, 128), lambda t: (t, 0, 0)),
            full((4 * LD, 1)),
            full((4 * LD, 1)),
            full((6 * LD, 1)),
            full((6 * LD, 1)),
            full((LD, LD)),
        ],
        out_specs=[
            pl.BlockSpec((1, 7 * LD, B), lambda t, s=s: (s * TS + t, 0, 0)),
            pl.BlockSpec((1, 6 * LD, B), lambda t, s=s: (s * TS + t, 0, 0)),
        ],
        out_shape=[
            jax.ShapeDtypeStruct((T, 7 * LD, B), jnp.float32),
            jax.ShapeDtypeStruct((T, 6 * LD, B), jnp.float32),
        ],
        input_output_aliases=aliases,
    )(*alias_args, xpad, g0, g1, g2, wk, bk, wo, bob, eye)


def kernel(inputs, se0, se1, se2, se3, ke0, ke1, ke2, Wr, br, Wo, bo):
    inT = jnp.transpose(inputs, (1, 2, 0))
    xpad = jnp.concatenate(
        [inT[:, 0:6, :], jnp.zeros((T, 2, B), jnp.float32)], axis=1)

    kidxT = inT[:, 10:13, :].astype(jnp.int32)

    sidxb = inputs[:, 0, 6:10].astype(jnp.int32)
    planes = [
        (jnp.arange(LD, dtype=jnp.int32)[:, None] * SVOCAB[j] + sidxb[None, :, j])
        for j in range(4)
    ]
    sidx64 = jnp.stack(planes, axis=0).reshape(4 * LD * B)

    sefs = [t.T.reshape(-1) for t in (se0, se1, se2, se3)]

    wk = Wr.reshape(4 * LD, 1)
    bk = br.reshape(4 * LD, 1)
    wo = Wo.reshape(6 * LD, 1)
    bob = bo.reshape(6 * LD, 1)
    eye = jnp.asarray(np.eye(LD, dtype=np.float32))

    stage_g = []
    gs = None
    for s in range(NSTAGE):
        ks = kidxT[s * TS:(s + 1) * TS]
        idxs = []
        for i in range(3):
            idxs.append(ks[:, i, 0:HB].reshape(ROWS_S))
            idxs.append(ks[:, i, HB:B].reshape(ROWS_S))
        outs = _sc_stage(s == NSTAGE - 1, ke0, ke1, ke2, idxs, sefs, sidx64)
        if s == NSTAGE - 1:
            g0s, g1s, g2s, gs = outs
        else:
            g0s, g1s, g2s = outs
        stage_g.append((g0s.reshape(TS, HB, 128),
                        g1s.reshape(TS, HB, 128),
                        g2s.reshape(TS, HB, 128)))

    prev = None
    for s in range(NSTAGE):
        g0s, g1s, g2s = stage_g[s]
        prev = _tc_stage(s, prev, xpad, g0s, g1s, g2s, wk, bk, wo, bob, eye)
    ok_p, oo_p = prev

    static = jnp.transpose(gs.reshape(4, LD, B), (2, 0, 1))
    known = jnp.transpose(ok_p.reshape(T, 7, LD, B), (3, 0, 2, 1))
    observed = jnp.transpose(oo_p.reshape(T, 6, LD, B), (3, 0, 2, 1))
    return (static, known, observed)
